# baseline (speedup 1.0000x reference)
import functools

import jax
import jax.numpy as jnp
from jax.experimental import pallas as pl
from jax.experimental.pallas import tpu as pltpu


def _rup(x, m):
    return ((x + m - 1) // m) * m


def _gemm_kernel(a_ref, b_ref, bias_ref, o_ref, *, relu):
    acc = jnp.dot(a_ref[...], b_ref[...], preferred_element_type=jnp.float32)
    acc = acc + bias_ref[...]
    if relu:
        acc = jnp.maximum(acc, 0.0)
    o_ref[...] = acc.astype(o_ref.dtype)


def _gemm(a, b, bias, relu=False, out_dtype=jnp.float32):
    M, K = a.shape
    K2, N = b.shape
    assert K == K2
    Np = _rup(N, 128)
    tn = Np if Np <= 512 else 512
    tm = min(512, _rup(M, 8))
    Kp = _rup(K, 128)
    Mp = _rup(M, tm)
    assert Kp * tn * 2 <= 12 * 1024 * 1024, "K too large for single-step GEMM"

    a_p = a.astype(jnp.bfloat16)
    if (Mp, Kp) != (M, K):
        a_p = jnp.pad(a_p, ((0, Mp - M), (0, Kp - K)))
    b_p = b.astype(jnp.bfloat16)
    if (Kp, Np) != (K, N):
        b_p = jnp.pad(b_p, ((0, Kp - K), (0, Np - N)))
    bias_p = bias.astype(jnp.float32)
    if Np != N:
        bias_p = jnp.pad(bias_p, (0, Np - N))
    bias_p = bias_p.reshape(1, Np)

    out = pl.pallas_call(
        functools.partial(_gemm_kernel, relu=relu),
        out_shape=jax.ShapeDtypeStruct((Mp, Np), out_dtype),
        grid=(Mp // tm, Np // tn),
        in_specs=[pl.BlockSpec((tm, Kp), lambda i, j: (i, 0)),
                  pl.BlockSpec((Kp, tn), lambda i, j: (0, j)),
                  pl.BlockSpec((1, tn), lambda i, j: (0, j))],
        out_specs=pl.BlockSpec((tm, tn), lambda i, j: (i, j)),
        compiler_params=pltpu.CompilerParams(
            dimension_semantics=("parallel", "parallel")),
    )(a_p, b_p, bias_p)
    if (Mp, Np) != (M, N):
        out = out[:M, :N]
    return out


def _pool3x3s2(y):
    bb, OH, OW, C = y.shape
    P = (OH - 3) // 2 + 1
    Q = (OW - 3) // 2 + 1
    if OH % 2:
        y = jnp.concatenate([y, jnp.zeros((bb, 1, OW, C), y.dtype)], axis=1)
    y = y.reshape(bb, (OH + 1) // 2, 2, OW, C)
    ev, od = y[:, :, 0], y[:, :, 1]
    v = jnp.maximum(jnp.maximum(ev[:, :P], od[:, :P]), ev[:, 1:P + 1])
    if OW % 2:
        v = jnp.concatenate([v, jnp.zeros((bb, P, 1, C), v.dtype)], axis=2)
    v = v.reshape(bb, P, (OW + 1) // 2, 2, C)
    ev, od = v[:, :, :, 0], v[:, :, :, 1]
    return jnp.maximum(jnp.maximum(ev[:, :, :Q], od[:, :, :Q]),
                       ev[:, :, 1:Q + 1])


def _conv3x3_kernel(x_ref, w_ref, b_ref, o_ref, *, bb, HP, WP, OH, OW, C,
                    pool, kgroup, nchw_flat, ring):
    M2 = bb * HP * WP
    Me = M2 - 2 * WP - 2
    X2 = x_ref[...]
    if kgroup == 9:
        a = jnp.concatenate(
            [X2[th * WP + tw:th * WP + tw + Me]
             for th in range(3) for tw in range(3)], axis=1)
        acc = jnp.dot(a, w_ref[...], preferred_element_type=jnp.float32)
    else:
        acc = None
        for th in range(3):
            for tw in range(3):
                off = th * WP + tw
                t = th * 3 + tw
                part = jnp.dot(X2[off:off + Me], w_ref[t * C:(t + 1) * C],
                               preferred_element_type=jnp.float32)
                acc = part if acc is None else acc + part
    y = jnp.maximum(acc + b_ref[...], 0.0).astype(jnp.bfloat16)
    y = jnp.concatenate(
        [y, jnp.zeros((M2 - Me, y.shape[1]), y.dtype)], axis=0)
    g = y.reshape(bb, HP, WP, y.shape[1])
    if pool:
        P, Q = (OH - 3) // 2 + 1, (OW - 3) // 2 + 1
        g2 = g[:, :2 * (P + 1)].reshape(bb, P + 1, 2, WP, g.shape[3])
        ev, od = g2[:, :, 0], g2[:, :, 1]
        v = jnp.maximum(jnp.maximum(ev[:, :P], od[:, :P]), ev[:, 1:P + 1])
        v2 = v[:, :, :2 * (Q + 1)].reshape(bb, P, Q + 1, 2, g.shape[3])
        ec, oc = v2[:, :, :, 0], v2[:, :, :, 1]
        r = jnp.maximum(jnp.maximum(ec[:, :, :Q], oc[:, :, :Q]),
                        ec[:, :, 1:Q + 1])
        if nchw_flat:
            r = r.transpose(0, 3, 1, 2).reshape(bb, -1)
    else:
        r = g[:, :OH, :OW, :]
    if ring != (0, 0, 0, 0):
        pt, pb, pleft, pright = ring
        Co = r.shape[3]
        rh, rw = r.shape[1], r.shape[2]
        z = lambda *sh: jnp.zeros(sh, r.dtype)
        r = jnp.concatenate(
            [z(bb, pt, rw, Co), r, z(bb, pb, rw, Co)], axis=1)
        r = jnp.concatenate(
            [z(bb, rh + pt + pb, pleft, Co), r,
             z(bb, rh + pt + pb, pright, Co)], axis=2)
    o_ref[...] = r


def _conv3x3(x2, HP, WP, w_km, bvec, pool, bb=8, OH=None, OW=None,
             kgroup=1, nchw_flat=False, ring=(0, 0, 0, 0)):
    Mtot, C = x2.shape
    B = Mtot // (HP * WP)
    OH = HP - 2 if OH is None else OH
    OW = WP - 2 if OW is None else OW
    Cout = w_km.shape[1]
    if pool:
        RH, RW = (OH - 3) // 2 + 1, (OW - 3) // 2 + 1
    else:
        RH, RW = OH, OW
    RH += ring[0] + ring[1]
    RW += ring[2] + ring[3]
    if nchw_flat:
        out_shape = jax.ShapeDtypeStruct((B, RH * RW * Cout), jnp.bfloat16)
        out_spec = pl.BlockSpec((bb, RH * RW * Cout), lambda i: (i, 0))
    else:
        out_shape = jax.ShapeDtypeStruct((B, RH, RW, Cout), jnp.bfloat16)
        out_spec = pl.BlockSpec((bb, RH, RW, Cout), lambda i: (i, 0, 0, 0))
    return pl.pallas_call(
        functools.partial(_conv3x3_kernel, bb=bb, HP=HP, WP=WP, OH=OH,
                          OW=OW, C=C, pool=pool, kgroup=kgroup,
                          nchw_flat=nchw_flat, ring=ring),
        out_shape=out_shape,
        grid=(B // bb,),
        in_specs=[pl.BlockSpec((bb * HP * WP, C), lambda i: (i, 0)),
                  pl.BlockSpec((9 * C, Cout), lambda i: (0, 0)),
                  pl.BlockSpec((1, Cout), lambda i: (0, 0))],
        out_specs=out_spec,
        compiler_params=pltpu.CompilerParams(
            dimension_semantics=("parallel",)),
    )(x2, w_km, bvec.astype(jnp.float32).reshape(1, Cout))


def _lstm_step(gates, c, H):
    i = jax.nn.sigmoid(gates[:, 0:H])
    f = jax.nn.sigmoid(gates[:, H:2 * H])
    g = jnp.tanh(gates[:, 2 * H:3 * H])
    o = jax.nn.sigmoid(gates[:, 3 * H:4 * H])
    c2 = f * c + i * g
    h2 = o * jnp.tanh(c2)
    return h2, c2


def _bilstm0_kernel(emb_ref, wih_ref, b_ref, whh_ref, o_ref, *, T, B, H):
    d = pl.program_id(0)
    whh = whh_ref[0]
    G = 4 * H
    xw = jnp.dot(emb_ref[...], wih_ref[0],
                 preferred_element_type=jnp.float32) + b_ref[0]
    xw3 = xw.reshape(T, B, G)

    @pl.when(d == 0)
    def _fwd():
        h = jnp.zeros((B, H), jnp.float32)
        c = jnp.zeros((B, H), jnp.float32)
        for t in range(T):
            gates = xw3[t] + jnp.dot(
                h.astype(jnp.bfloat16), whh, preferred_element_type=jnp.float32)
            h, c = _lstm_step(gates, c, H)
            o_ref[0, t] = h.astype(jnp.bfloat16)

    @pl.when(d == 1)
    def _bwd():
        h = jnp.zeros((B, H), jnp.float32)
        c = jnp.zeros((B, H), jnp.float32)
        for t in range(T - 1, -1, -1):
            gates = xw3[t] + jnp.dot(
                h.astype(jnp.bfloat16), whh, preferred_element_type=jnp.float32)
            h, c = _lstm_step(gates, c, H)
            o_ref[0, t] = h.astype(jnp.bfloat16)


def _bilstm1_kernel(x_ref, wih_ref, b_ref, whh_ref, w1_ref, o_ref, *,
                    T, B, H):
    whh = whh_ref[...]
    G = 4 * H
    xw = jnp.dot(x_ref[...], wih_ref[0],
                 preferred_element_type=jnp.float32) + b_ref[0]
    xw3 = xw.reshape(T, B, G)
    h = jnp.zeros((B, H), jnp.float32)
    c = jnp.zeros((B, H), jnp.float32)
    for t in range(T):
        gates = xw3[t] + jnp.dot(
            h.astype(jnp.bfloat16), whh, preferred_element_type=jnp.float32)
        h, c = _lstm_step(gates, c, H)
    gates_b = jnp.dot(x_ref[(T - 1) * B:T * B], wih_ref[1],
                      preferred_element_type=jnp.float32) + b_ref[1]
    hb, _ = _lstm_step(gates_b, jnp.zeros((B, H), jnp.float32), H)
    hcat = jnp.concatenate([h, hb], axis=1)
    tf = jnp.sum(hcat * w1_ref[...], axis=1, keepdims=True)
    col = jax.lax.broadcasted_iota(jnp.int32, (B, 128), 1)
    o_ref[...] = jnp.where(col == 0, tf, 0.0)


def _run_bilstm(emb_tb, wih_s0, b_s0, whh_cat0, wih_s1, b_s1,
                whh1_f, fc1_w):
    T, B, E = emb_tb.shape
    H = whh1_f.shape[0]
    G = 4 * H
    emb2 = emb_tb.reshape(T * B, E).astype(jnp.bfloat16)
    hs = pl.pallas_call(
        functools.partial(_bilstm0_kernel, T=T, B=B, H=H),
        out_shape=jax.ShapeDtypeStruct((2, T, B, H), jnp.bfloat16),
        grid=(2,),
        in_specs=[pl.BlockSpec((T * B, E), lambda d: (0, 0)),
                  pl.BlockSpec((1, E, G), lambda d: (d, 0, 0)),
                  pl.BlockSpec((1, 1, G), lambda d: (d, 0, 0)),
                  pl.BlockSpec((1, H, G), lambda d: (d, 0, 0))],
        out_specs=pl.BlockSpec((1, T, B, H), lambda d: (d, 0, 0, 0)),
        compiler_params=pltpu.CompilerParams(
            dimension_semantics=("parallel",)),
    )(emb2, wih_s0, b_s0, whh_cat0)
    inp1 = hs.transpose(1, 2, 0, 3).reshape(T * B, 2 * H)
    out = pl.pallas_call(
        functools.partial(_bilstm1_kernel, T=T, B=B, H=H),
        out_shape=jax.ShapeDtypeStruct((B, 128), jnp.float32),
        grid=(1,),
        in_specs=[pl.BlockSpec((T * B, 2 * H), lambda i: (0, 0)),
                  pl.BlockSpec((2, 2 * H, G), lambda i: (0, 0, 0)),
                  pl.BlockSpec((2, 1, G), lambda i: (0, 0, 0)),
                  pl.BlockSpec((H, G), lambda i: (0, 0)),
                  pl.BlockSpec((1, 2 * H), lambda i: (0, 0))],
        out_specs=pl.BlockSpec((B, 128), lambda i: (0, 0)),
        compiler_params=pltpu.CompilerParams(
            dimension_semantics=("arbitrary",)),
    )(inp1, wih_s1, b_s1, whh1_f, fc1_w.reshape(1, 2 * H).astype(jnp.float32))
    return out[:, 0:1]


def kernel(token_ids, seq_len, image, embedding,
           lstm_l0_d0_wih, lstm_l0_d0_whh, lstm_l0_d0_b,
           lstm_l0_d1_wih, lstm_l0_d1_whh, lstm_l0_d1_b,
           lstm_l1_d0_wih, lstm_l1_d0_whh, lstm_l1_d0_b,
           lstm_l1_d1_wih, lstm_l1_d1_whh, lstm_l1_d1_b,
           conv1_w, conv1_b, conv2_w, conv2_b, conv3_w, conv3_b,
           conv4_w, conv4_b, conv5_w, conv5_b,
           fc1_w, fc1_b, cls1_w, cls1_b, cls2_w, cls2_b,
           cls3_w, cls3_b, fc2_w, fc2_b):
    emb_tb = embedding[token_ids.T]
    wih_s0 = jnp.stack([lstm_l0_d0_wih, lstm_l0_d1_wih])
    b_s0 = jnp.stack([lstm_l0_d0_b, lstm_l0_d1_b]).reshape(2, 1, -1)
    whh_cat0 = jnp.stack([lstm_l0_d0_whh, lstm_l0_d1_whh]).astype(jnp.bfloat16)
    wih_s1 = jnp.stack([lstm_l1_d0_wih, lstm_l1_d1_wih])
    b_s1 = jnp.stack([lstm_l1_d0_b, lstm_l1_d1_b]).reshape(2, 1, -1)
    text_feat = _run_bilstm(emb_tb, wih_s0, b_s0, whh_cat0,
                            wih_s1, b_s1,
                            lstm_l1_d0_whh.astype(jnp.bfloat16), fc1_w)
    text_feat = (text_feat + fc1_b).astype(jnp.bfloat16)

    B = image.shape[0]
    xp = jnp.pad(image.astype(jnp.bfloat16),
                 ((0, 0), (0, 1), (2, 30), (2, 30)))
    x = xp.reshape(B, 4, 64, 4, 64, 4).transpose(0, 2, 4, 3, 5, 1)
    x = x.reshape(B, 64, 64, 64)
    w1 = conv1_w.reshape(11, 11, 3, 64)
    w1 = jnp.pad(w1, ((0, 1), (0, 1), (0, 1), (0, 0)))
    w1 = w1.reshape(3, 4, 3, 4, 4, 64).transpose(0, 2, 1, 3, 4, 5)
    w1 = w1.reshape(9 * 64, 64)

    x = _conv3x3(x.reshape(-1, 64), 64, 64, w1, conv1_b, pool=True,
                 bb=2, OH=55, OW=55, kgroup=9,
                 ring=(2, 3, 2, 3))
    x = _conv3x3(x.reshape(-1, 64), 32, 32, conv2_w, conv2_b, pool=True,
                 OH=29, OW=29, kgroup=9,
                 ring=(1, 1, 1, 1))
    x = _conv3x3(x.reshape(-1, 192), 16, 16, conv3_w, conv3_b, pool=False,
                 ring=(1, 1, 1, 1))
    x = _conv3x3(x.reshape(-1, 384), 16, 16, conv4_w, conv4_b, pool=False,
                 ring=(1, 1, 1, 1))
    x = _conv3x3(x.reshape(-1, 256), 16, 16, conv5_w, conv5_b, pool=True,
                 nchw_flat=True)

    x = _gemm(x, cls1_w, cls1_b, relu=True, out_dtype=jnp.bfloat16)
    x = _gemm(x, cls2_w, cls2_b, relu=True, out_dtype=jnp.bfloat16)
    x = _gemm(x, cls3_w, cls3_b, relu=False, out_dtype=jnp.bfloat16)

    out = _gemm(jnp.concatenate([x, text_feat], axis=1), fc2_w, fc2_b)
    return out


# speedup vs baseline: 1.0464x; 1.0464x over previous
import functools

import jax
import jax.numpy as jnp
from jax.experimental import pallas as pl
from jax.experimental.pallas import tpu as pltpu


def _rup(x, m):
    return ((x + m - 1) // m) * m


def _gemm_kernel(a_ref, b_ref, bias_ref, o_ref, *, relu):
    acc = jnp.dot(a_ref[...], b_ref[...], preferred_element_type=jnp.float32)
    acc = acc + bias_ref[...]
    if relu:
        acc = jnp.maximum(acc, 0.0)
    o_ref[...] = acc.astype(o_ref.dtype)


def _gemm(a, b, bias, relu=False, out_dtype=jnp.float32):
    M, K = a.shape
    K2, N = b.shape
    assert K == K2
    Np = _rup(N, 128)
    tn = Np if Np <= 512 else 512
    tm = min(512, _rup(M, 8))
    Kp = _rup(K, 128)
    Mp = _rup(M, tm)
    assert Kp * tn * 2 <= 12 * 1024 * 1024, "K too large for single-step GEMM"

    a_p = a.astype(jnp.bfloat16)
    if (Mp, Kp) != (M, K):
        a_p = jnp.pad(a_p, ((0, Mp - M), (0, Kp - K)))
    b_p = b.astype(jnp.bfloat16)
    if (Kp, Np) != (K, N):
        b_p = jnp.pad(b_p, ((0, Kp - K), (0, Np - N)))
    bias_p = bias.astype(jnp.float32)
    if Np != N:
        bias_p = jnp.pad(bias_p, (0, Np - N))
    bias_p = bias_p.reshape(1, Np)

    out = pl.pallas_call(
        functools.partial(_gemm_kernel, relu=relu),
        out_shape=jax.ShapeDtypeStruct((Mp, Np), out_dtype),
        grid=(Mp // tm, Np // tn),
        in_specs=[pl.BlockSpec((tm, Kp), lambda i, j: (i, 0)),
                  pl.BlockSpec((Kp, tn), lambda i, j: (0, j)),
                  pl.BlockSpec((1, tn), lambda i, j: (0, j))],
        out_specs=pl.BlockSpec((tm, tn), lambda i, j: (i, j)),
        compiler_params=pltpu.CompilerParams(
            dimension_semantics=("parallel", "parallel")),
    )(a_p, b_p, bias_p)
    if (Mp, Np) != (M, N):
        out = out[:M, :N]
    return out


def _pool3x3s2(y):
    bb, OH, OW, C = y.shape
    P = (OH - 3) // 2 + 1
    Q = (OW - 3) // 2 + 1
    if OH % 2:
        y = jnp.concatenate([y, jnp.zeros((bb, 1, OW, C), y.dtype)], axis=1)
    y = y.reshape(bb, (OH + 1) // 2, 2, OW, C)
    ev, od = y[:, :, 0], y[:, :, 1]
    v = jnp.maximum(jnp.maximum(ev[:, :P], od[:, :P]), ev[:, 1:P + 1])
    if OW % 2:
        v = jnp.concatenate([v, jnp.zeros((bb, P, 1, C), v.dtype)], axis=2)
    v = v.reshape(bb, P, (OW + 1) // 2, 2, C)
    ev, od = v[:, :, :, 0], v[:, :, :, 1]
    return jnp.maximum(jnp.maximum(ev[:, :, :Q], od[:, :, :Q]),
                       ev[:, :, 1:Q + 1])


def _conv3x3_kernel(x_ref, w_ref, b_ref, o_ref, *, bb, HP, WP, OH, OW, C,
                    pool, kgroup, nchw_flat, ring):
    M2 = bb * HP * WP
    Me = M2 - 2 * WP - 2
    X2 = x_ref[...]
    if kgroup == 9:
        a = jnp.concatenate(
            [X2[th * WP + tw:th * WP + tw + Me]
             for th in range(3) for tw in range(3)], axis=1)
        acc = jnp.dot(a, w_ref[...], preferred_element_type=jnp.float32)
    else:
        acc = None
        for th in range(3):
            for tw in range(3):
                off = th * WP + tw
                t = th * 3 + tw
                part = jnp.dot(X2[off:off + Me], w_ref[t * C:(t + 1) * C],
                               preferred_element_type=jnp.float32)
                acc = part if acc is None else acc + part
    y = jnp.maximum(acc + b_ref[...], 0.0).astype(jnp.bfloat16)
    y = jnp.concatenate(
        [y, jnp.zeros((M2 - Me, y.shape[1]), y.dtype)], axis=0)
    g = y.reshape(bb, HP, WP, y.shape[1])
    if pool:
        P, Q = (OH - 3) // 2 + 1, (OW - 3) // 2 + 1
        g2 = g[:, :2 * (P + 1)].reshape(bb, P + 1, 2, WP, g.shape[3])
        ev, od = g2[:, :, 0], g2[:, :, 1]
        v = jnp.maximum(jnp.maximum(ev[:, :P], od[:, :P]), ev[:, 1:P + 1])
        v2 = v[:, :, :2 * (Q + 1)].reshape(bb, P, Q + 1, 2, g.shape[3])
        ec, oc = v2[:, :, :, 0], v2[:, :, :, 1]
        r = jnp.maximum(jnp.maximum(ec[:, :, :Q], oc[:, :, :Q]),
                        ec[:, :, 1:Q + 1])
        if nchw_flat:
            r = r.transpose(0, 3, 1, 2).reshape(bb, -1)
    else:
        r = g[:, :OH, :OW, :]
    if ring != (0, 0, 0, 0):
        pt, pb, pleft, pright = ring
        Co = r.shape[3]
        rh, rw = r.shape[1], r.shape[2]
        z = lambda *sh: jnp.zeros(sh, r.dtype)
        r = jnp.concatenate(
            [z(bb, pt, rw, Co), r, z(bb, pb, rw, Co)], axis=1)
        r = jnp.concatenate(
            [z(bb, rh + pt + pb, pleft, Co), r,
             z(bb, rh + pt + pb, pright, Co)], axis=2)
    o_ref[...] = r


def _conv3x3(x2, HP, WP, w_km, bvec, pool, bb=8, OH=None, OW=None,
             kgroup=1, nchw_flat=False, ring=(0, 0, 0, 0)):
    Mtot, C = x2.shape
    B = Mtot // (HP * WP)
    OH = HP - 2 if OH is None else OH
    OW = WP - 2 if OW is None else OW
    Cout = w_km.shape[1]
    if pool:
        RH, RW = (OH - 3) // 2 + 1, (OW - 3) // 2 + 1
    else:
        RH, RW = OH, OW
    RH += ring[0] + ring[1]
    RW += ring[2] + ring[3]
    if nchw_flat:
        out_shape = jax.ShapeDtypeStruct((B, RH * RW * Cout), jnp.bfloat16)
        out_spec = pl.BlockSpec((bb, RH * RW * Cout), lambda i: (i, 0))
    else:
        out_shape = jax.ShapeDtypeStruct((B, RH, RW, Cout), jnp.bfloat16)
        out_spec = pl.BlockSpec((bb, RH, RW, Cout), lambda i: (i, 0, 0, 0))
    return pl.pallas_call(
        functools.partial(_conv3x3_kernel, bb=bb, HP=HP, WP=WP, OH=OH,
                          OW=OW, C=C, pool=pool, kgroup=kgroup,
                          nchw_flat=nchw_flat, ring=ring),
        out_shape=out_shape,
        grid=(B // bb,),
        in_specs=[pl.BlockSpec((bb * HP * WP, C), lambda i: (i, 0)),
                  pl.BlockSpec((9 * C, Cout), lambda i: (0, 0)),
                  pl.BlockSpec((1, Cout), lambda i: (0, 0))],
        out_specs=out_spec,
        compiler_params=pltpu.CompilerParams(
            dimension_semantics=("parallel",)),
    )(x2, w_km, bvec.astype(jnp.float32).reshape(1, Cout))


def _lstm_step(gates, c, H):
    i = jax.nn.sigmoid(gates[:, 0:H])
    f = jax.nn.sigmoid(gates[:, H:2 * H])
    g = jnp.tanh(gates[:, 2 * H:3 * H])
    o = jax.nn.sigmoid(gates[:, 3 * H:4 * H])
    c2 = f * c + i * g
    h2 = o * jnp.tanh(c2)
    return h2, c2


def _bilstm0_kernel(emb_ref, wih_ref, b_ref, whh_ref, o_ref, *, T, B, H):
    d = pl.program_id(0)
    whh = whh_ref[0]
    G = 4 * H
    xw = jnp.dot(emb_ref[...], wih_ref[0],
                 preferred_element_type=jnp.float32) + b_ref[0]
    xw3 = xw.reshape(T, B, G)

    @pl.when(d == 0)
    def _fwd():
        h = jnp.zeros((B, H), jnp.float32)
        c = jnp.zeros((B, H), jnp.float32)
        for t in range(T):
            gates = xw3[t] + jnp.dot(
                h.astype(jnp.bfloat16), whh, preferred_element_type=jnp.float32)
            h, c = _lstm_step(gates, c, H)
            o_ref[0, t] = h.astype(jnp.bfloat16)

    @pl.when(d == 1)
    def _bwd():
        h = jnp.zeros((B, H), jnp.float32)
        c = jnp.zeros((B, H), jnp.float32)
        for t in range(T - 1, -1, -1):
            gates = xw3[t] + jnp.dot(
                h.astype(jnp.bfloat16), whh, preferred_element_type=jnp.float32)
            h, c = _lstm_step(gates, c, H)
            o_ref[0, t] = h.astype(jnp.bfloat16)


def _bilstm1_kernel(x_ref, wih_ref, b_ref, whh_ref, w1_ref, o_ref, *,
                    T, B, H):
    whh = whh_ref[...]
    G = 4 * H
    xw = jnp.dot(x_ref[...], wih_ref[0],
                 preferred_element_type=jnp.float32) + b_ref[0]
    xw3 = xw.reshape(T, B, G)
    h = jnp.zeros((B, H), jnp.float32)
    c = jnp.zeros((B, H), jnp.float32)
    for t in range(T):
        gates = xw3[t] + jnp.dot(
            h.astype(jnp.bfloat16), whh, preferred_element_type=jnp.float32)
        h, c = _lstm_step(gates, c, H)
    gates_b = jnp.dot(x_ref[(T - 1) * B:T * B], wih_ref[1],
                      preferred_element_type=jnp.float32) + b_ref[1]
    hb, _ = _lstm_step(gates_b, jnp.zeros((B, H), jnp.float32), H)
    hcat = jnp.concatenate([h, hb], axis=1)
    tf = jnp.sum(hcat * w1_ref[...], axis=1, keepdims=True)
    col = jax.lax.broadcasted_iota(jnp.int32, (B, 128), 1)
    o_ref[...] = jnp.where(col == 0, tf, 0.0)


def _run_bilstm(emb_tb, wih_s0, b_s0, whh_cat0, wih_s1, b_s1,
                whh1_f, fc1_w):
    T, B, E = emb_tb.shape
    H = whh1_f.shape[0]
    G = 4 * H
    emb2 = emb_tb.reshape(T * B, E).astype(jnp.bfloat16)
    hs = pl.pallas_call(
        functools.partial(_bilstm0_kernel, T=T, B=B, H=H),
        out_shape=jax.ShapeDtypeStruct((2, T, B, H), jnp.bfloat16),
        grid=(2,),
        in_specs=[pl.BlockSpec((T * B, E), lambda d: (0, 0)),
                  pl.BlockSpec((1, E, G), lambda d: (d, 0, 0)),
                  pl.BlockSpec((1, 1, G), lambda d: (d, 0, 0)),
                  pl.BlockSpec((1, H, G), lambda d: (d, 0, 0))],
        out_specs=pl.BlockSpec((1, T, B, H), lambda d: (d, 0, 0, 0)),
        compiler_params=pltpu.CompilerParams(
            dimension_semantics=("parallel",)),
    )(emb2, wih_s0, b_s0, whh_cat0)
    inp1 = hs.transpose(1, 2, 0, 3).reshape(T * B, 2 * H)
    out = pl.pallas_call(
        functools.partial(_bilstm1_kernel, T=T, B=B, H=H),
        out_shape=jax.ShapeDtypeStruct((B, 128), jnp.float32),
        grid=(1,),
        in_specs=[pl.BlockSpec((T * B, 2 * H), lambda i: (0, 0)),
                  pl.BlockSpec((2, 2 * H, G), lambda i: (0, 0, 0)),
                  pl.BlockSpec((2, 1, G), lambda i: (0, 0, 0)),
                  pl.BlockSpec((H, G), lambda i: (0, 0)),
                  pl.BlockSpec((1, 2 * H), lambda i: (0, 0))],
        out_specs=pl.BlockSpec((B, 128), lambda i: (0, 0)),
        compiler_params=pltpu.CompilerParams(
            dimension_semantics=("arbitrary",)),
    )(inp1, wih_s1, b_s1, whh1_f, fc1_w.reshape(1, 2 * H).astype(jnp.float32))
    return out[:, 0:1]


def kernel(token_ids, seq_len, image, embedding,
           lstm_l0_d0_wih, lstm_l0_d0_whh, lstm_l0_d0_b,
           lstm_l0_d1_wih, lstm_l0_d1_whh, lstm_l0_d1_b,
           lstm_l1_d0_wih, lstm_l1_d0_whh, lstm_l1_d0_b,
           lstm_l1_d1_wih, lstm_l1_d1_whh, lstm_l1_d1_b,
           conv1_w, conv1_b, conv2_w, conv2_b, conv3_w, conv3_b,
           conv4_w, conv4_b, conv5_w, conv5_b,
           fc1_w, fc1_b, cls1_w, cls1_b, cls2_w, cls2_b,
           cls3_w, cls3_b, fc2_w, fc2_b):
    emb_tb = embedding[token_ids.T]
    wih_s0 = jnp.stack([lstm_l0_d0_wih, lstm_l0_d1_wih])
    b_s0 = jnp.stack([lstm_l0_d0_b, lstm_l0_d1_b]).reshape(2, 1, -1)
    whh_cat0 = jnp.stack([lstm_l0_d0_whh, lstm_l0_d1_whh]).astype(jnp.bfloat16)
    wih_s1 = jnp.stack([lstm_l1_d0_wih, lstm_l1_d1_wih])
    b_s1 = jnp.stack([lstm_l1_d0_b, lstm_l1_d1_b]).reshape(2, 1, -1)
    text_feat = _run_bilstm(emb_tb, wih_s0, b_s0, whh_cat0,
                            wih_s1, b_s1,
                            lstm_l1_d0_whh.astype(jnp.bfloat16), fc1_w)
    text_feat = (text_feat + fc1_b).astype(jnp.bfloat16)

    B = image.shape[0]
    xp = jnp.pad(image.astype(jnp.bfloat16),
                 ((0, 0), (0, 0), (2, 30), (2, 30)))
    x = xp.reshape(B, 3, 64, 4, 64, 4).transpose(0, 2, 4, 3, 5, 1)
    x = x.reshape(B, 64, 64, 48)
    w1 = conv1_w.reshape(11, 11, 3, 64)
    w1 = jnp.pad(w1, ((0, 1), (0, 1), (0, 0), (0, 0)))
    w1 = w1.reshape(3, 4, 3, 4, 3, 64).transpose(0, 2, 1, 3, 4, 5)
    w1 = w1.reshape(9 * 48, 64)

    x = _conv3x3(x.reshape(-1, 48), 64, 64, w1, conv1_b, pool=True,
                 bb=2, OH=55, OW=55, kgroup=9,
                 ring=(2, 3, 2, 3))
    x = _conv3x3(x.reshape(-1, 64), 32, 32, conv2_w, conv2_b, pool=True,
                 OH=29, OW=29, kgroup=9,
                 ring=(1, 1, 1, 1))
    x = _conv3x3(x.reshape(-1, 192), 16, 16, conv3_w, conv3_b, pool=False,
                 ring=(1, 1, 1, 1))
    x = _conv3x3(x.reshape(-1, 384), 16, 16, conv4_w, conv4_b, pool=False,
                 ring=(1, 1, 1, 1))
    x = _conv3x3(x.reshape(-1, 256), 16, 16, conv5_w, conv5_b, pool=True,
                 nchw_flat=True)

    x = _gemm(x, cls1_w, cls1_b, relu=True, out_dtype=jnp.bfloat16)
    x = _gemm(x, cls2_w, cls2_b, relu=True, out_dtype=jnp.bfloat16)
    x = _gemm(x, cls3_w, cls3_b, relu=False, out_dtype=jnp.bfloat16)

    out = _gemm(jnp.concatenate([x, text_feat], axis=1), fc2_w, fc2_b)
    return out
